# Initial kernel scaffold; baseline (speedup 1.0000x reference)
import jax
import jax.numpy as jnp
from jax.experimental import pallas as pl


def kernel(x, edge_index, W1a, b1a, W1b, b1b, W2a, b2a, W2b, b2b, W3a, b3a, W3b, b3b, W4a, b4a, W4b, b4b):
    raise NotImplementedError("write your pallas kernel here")


# speedup vs baseline: 6.9062x; 6.9062x over previous
import functools

import jax
import jax.numpy as jnp
from jax import lax
from jax.experimental import pallas as pl
from jax.experimental.pallas import tpu as pltpu
from jax.experimental.pallas import tpu_sc as plsc

NT = 16
NC = 2
B = 128
KB = 4
N_ACC = 100096


def _make_sc_agg(nchunk, n_rows, ept):
    cps = nchunk // NC
    nbatch = ept // (KB * B)
    rpt = N_ACC // NT
    scratch = (
        [pltpu.VMEM((B,), jnp.int32) for _ in range(2 * KB)]
        + [pltpu.VMEM((B, 16), jnp.float32) for _ in range(KB)]
        + [pltpu.VMEM_SHARED((N_ACC, 16), jnp.float32),
           pltpu.SemaphoreType.DMA, pltpu.SemaphoreType.DMA]
    )
    mesh = plsc.VectorSubcoreMesh(core_axis_name="c", subcore_axis_name="s")

    @functools.partial(
        pl.kernel, mesh=mesh,
        out_type=jax.ShapeDtypeStruct((nchunk, N_ACC, 16), jnp.float32),
        scratch_types=scratch,
        compiler_params=pltpu.CompilerParams(use_tc_tiling_on_sc=False))
    def sc_agg(tables, srcp, dstp, zrows, out, *rest):
        srcv = rest[0:KB]
        dstv = rest[KB:2 * KB]
        rowv = rest[2 * KB:3 * KB]
        acc, sem_i, sem_g = rest[3 * KB:3 * KB + 3]
        c = lax.axis_index("c")
        s = lax.axis_index("s")
        row0 = s * rpt
        ebase = s * ept
        for k in range(cps):
            chunk = c * cps + k
            pltpu.sync_copy(zrows.at[pl.ds(row0, rpt)], acc.at[pl.ds(row0, rpt)])
            plsc.subcore_barrier()

            def body(i, carry):
                base = ebase + i * (KB * B)
                hs = []
                for b in range(KB):
                    o = pl.multiple_of(base + b * B, B)
                    hs.append(pltpu.async_copy(srcp.at[pl.ds(o, B)], srcv[b], sem_i))
                    hs.append(pltpu.async_copy(dstp.at[pl.ds(o, B)], dstv[b], sem_i))
                for h in hs:
                    h.wait()
                gs = [pltpu.async_copy(tables.at[chunk].at[srcv[b]], rowv[b], sem_g)
                      for b in range(KB)]
                for g in gs:
                    g.wait()
                for b in range(KB):
                    pltpu.sync_copy(rowv[b], acc.at[dstv[b]], add=True)
                return carry

            lax.fori_loop(0, nbatch, body, 0)
            plsc.subcore_barrier()
            pltpu.sync_copy(acc.at[pl.ds(row0, rpt)],
                            out.at[chunk].at[pl.ds(row0, rpt)])
            plsc.subcore_barrier()

    return sc_agg


def _tc_mlp(x_ref, g_ref, wa, ba, wb, bb, o_ref):
    h = x_ref[...] + g_ref[...]
    a = jnp.maximum(jnp.dot(h, wa[...], preferred_element_type=jnp.float32) + ba[...], 0.0)
    o_ref[...] = jnp.dot(a, wb[...], preferred_element_type=jnp.float32) + bb[...]


def _tc_mlp_dec(t2, z_ref, g_ref, wa, ba, wb, bb, o_ref):
    h = z_ref[...] + g_ref[...]
    a = jnp.maximum(jnp.dot(h, wa[...], preferred_element_type=jnp.float32) + ba[...], 0.0)
    scat = jnp.dot(a, wb[...], preferred_element_type=jnp.float32) + bb[...]
    zs = scat[:, :64]
    zt = scat[:, 64:]
    p = jnp.dot(zs, zt.T, preferred_element_type=jnp.float32)
    r = lax.broadcasted_iota(jnp.int32, (t2, t2), 0)
    cc = lax.broadcasted_iota(jnp.int32, (t2, t2), 1)
    pm = jnp.where((r // 9) == (cc // 9), p, 0.0)
    kc = lax.broadcasted_iota(jnp.int32, (t2, 16), 0) % 9
    kk = lax.broadcasted_iota(jnp.int32, (t2, 16), 1)
    fold = (kc == kk).astype(jnp.float32)
    o_ref[...] = jnp.dot(pm, fold, preferred_element_type=jnp.float32)


def _blockdiag(a, b):
    z = jnp.zeros((a.shape[0] + b.shape[0], a.shape[1] + b.shape[1]), jnp.float32)
    return z.at[:a.shape[0], :a.shape[1]].set(a).at[a.shape[0]:, a.shape[1]:].set(b)


def kernel(x, edge_index, W1a, b1a, W1b, b1b, W2a, b2a, W2b, b2b,
           W3a, b3a, W3b, b3b, W4a, b4a, W4b, b4b):
    n, in_dim = x.shape
    e = edge_index.shape[1]
    ept = -(-e // (NT * KB * B)) * (KB * B)
    e_pad = NT * ept

    src = edge_index[0]
    dst = edge_index[1]
    pad = e_pad - e
    srcp = jnp.concatenate([src, jnp.zeros((pad,), jnp.int32)])
    dstp = jnp.concatenate([dst, jnp.full((pad,), n, jnp.int32)])
    zrows = jnp.zeros((N_ACC, 16), jnp.float32)

    x32 = jnp.pad(x, ((0, 0), (0, 32 - in_dim)))
    xtab = x32.reshape(n, 2, 16).transpose(1, 0, 2)
    aggx3 = _make_sc_agg(2, n, ept)(xtab, srcp, dstp, zrows)
    aggx = aggx3[:, :n, :].transpose(1, 0, 2).reshape(n, 32)

    wa1 = jnp.concatenate(
        [jnp.pad(W1a, ((0, 2), (0, 0))), jnp.pad(W3a, ((0, 2), (0, 0)))], axis=1)
    ba1 = jnp.concatenate([b1a, b3a]).reshape(1, 128)
    wb1 = _blockdiag(W1b, W3b)
    bb1 = jnp.concatenate([b1b, b3b]).reshape(1, 128)
    t1 = 512
    g1 = -(-n // t1)
    zcat = pl.pallas_call(
        _tc_mlp,
        grid=(g1,),
        in_specs=[
            pl.BlockSpec((t1, 32), lambda i: (i, 0)),
            pl.BlockSpec((t1, 32), lambda i: (i, 0)),
            pl.BlockSpec((32, 128), lambda i: (0, 0)),
            pl.BlockSpec((1, 128), lambda i: (0, 0)),
            pl.BlockSpec((128, 128), lambda i: (0, 0)),
            pl.BlockSpec((1, 128), lambda i: (0, 0)),
        ],
        out_specs=pl.BlockSpec((t1, 128), lambda i: (i, 0)),
        out_shape=jax.ShapeDtypeStruct((n, 128), jnp.float32),
    )(x32, aggx, wa1, ba1, wb1, bb1)

    ztab = zcat.reshape(n, 8, 16).transpose(1, 0, 2)
    aggz3 = _make_sc_agg(8, n, ept)(ztab, srcp, dstp, zrows)
    aggz = aggz3[:, :n, :].transpose(1, 0, 2).reshape(n, 128)

    wa2 = _blockdiag(W2a, W4a)
    ba2 = jnp.concatenate([b2a, b4a]).reshape(1, 128)
    wb2 = _blockdiag(W2b, W4b)
    bb2 = jnp.concatenate([b2b, b4b]).reshape(1, 128)
    t2 = 576
    g2 = -(-n // t2)
    out16 = pl.pallas_call(
        functools.partial(_tc_mlp_dec, t2),
        grid=(g2,),
        in_specs=[
            pl.BlockSpec((t2, 128), lambda i: (i, 0)),
            pl.BlockSpec((t2, 128), lambda i: (i, 0)),
            pl.BlockSpec((128, 128), lambda i: (0, 0)),
            pl.BlockSpec((1, 128), lambda i: (0, 0)),
            pl.BlockSpec((128, 128), lambda i: (0, 0)),
            pl.BlockSpec((1, 128), lambda i: (0, 0)),
        ],
        out_specs=pl.BlockSpec((t2, 16), lambda i: (i, 0)),
        out_shape=jax.ShapeDtypeStruct((n, 16), jnp.float32),
    )(zcat, aggz, wa2, ba2, wb2, bb2)
    return out16[:, :9]


# speedup vs baseline: 8.9412x; 1.2947x over previous
import functools

import jax
import jax.numpy as jnp
from jax import lax
from jax.experimental import pallas as pl
from jax.experimental.pallas import tpu as pltpu
from jax.experimental.pallas import tpu_sc as plsc

NT = 16
NC = 2
B = 128
KB = 4
N_ACC = 100096


def _make_sc_agg(nchunk, n_rows, ept):
    cps = nchunk // NC
    nbatch = ept // (KB * B)
    rpt = N_ACC // NT
    lview = n_rows * nchunk - nchunk + 1
    scratch = (
        [pltpu.VMEM((B,), jnp.int32) for _ in range(2 * KB)]
        + [pltpu.VMEM((B, 16), jnp.float32) for _ in range(KB)]
        + [pltpu.VMEM_SHARED((N_ACC, 16), jnp.float32),
           pltpu.SemaphoreType.DMA, pltpu.SemaphoreType.DMA]
    )
    mesh = plsc.VectorSubcoreMesh(core_axis_name="c", subcore_axis_name="s")

    @functools.partial(
        pl.kernel, mesh=mesh,
        out_type=jax.ShapeDtypeStruct((N_ACC, nchunk, 16), jnp.float32),
        scratch_types=scratch,
        compiler_params=pltpu.CompilerParams(use_tc_tiling_on_sc=False))
    def sc_agg(tables, srcn, dstp, zrows, out, *rest):
        srcv = rest[0:KB]
        dstv = rest[KB:2 * KB]
        rowv = rest[2 * KB:3 * KB]
        acc, sem_i, sem_g = rest[3 * KB:3 * KB + 3]
        c = lax.axis_index("c")
        s = lax.axis_index("s")
        row0 = s * rpt
        ebase = s * ept
        for core_id in range(NC):
            @pl.when(c == core_id)
            def _():
                for k in range(cps):
                    chunk = core_id * cps + k
                    tview = tables.at[pl.ds(chunk, lview)]
                    pltpu.sync_copy(zrows.at[pl.ds(row0, rpt)],
                                    acc.at[pl.ds(row0, rpt)])
                    plsc.subcore_barrier()

                    def body(i, carry):
                        base = ebase + i * (KB * B)
                        hs = []
                        for b in range(KB):
                            o = pl.multiple_of(base + b * B, B)
                            hs.append(pltpu.async_copy(
                                srcn.at[pl.ds(o, B)], srcv[b], sem_i))
                            hs.append(pltpu.async_copy(
                                dstp.at[pl.ds(o, B)], dstv[b], sem_i))
                        for h in hs:
                            h.wait()
                        gs = [pltpu.async_copy(
                            tview.at[srcv[b]], rowv[b], sem_g)
                            for b in range(KB)]
                        for g in gs:
                            g.wait()
                        for b in range(KB):
                            pltpu.sync_copy(rowv[b], acc.at[dstv[b]], add=True)
                        return carry

                    lax.fori_loop(0, nbatch, body, 0)
                    plsc.subcore_barrier()
                    pltpu.sync_copy(acc.at[pl.ds(row0, rpt)],
                                    out.at[pl.ds(row0, rpt), chunk])
                    plsc.subcore_barrier()

    return sc_agg


def _tc_mlp(x_ref, g_ref, wa, ba, wb, bb, o_ref):
    h = x_ref[...] + g_ref[...]
    a = jnp.maximum(jnp.dot(h, wa[...], preferred_element_type=jnp.float32) + ba[...], 0.0)
    o_ref[...] = jnp.dot(a, wb[...], preferred_element_type=jnp.float32) + bb[...]


def _tc_mlp_dec(t2, z_ref, g_ref, wa, ba, wb, bb, o_ref):
    h = z_ref[...] + g_ref[...]
    a = jnp.maximum(jnp.dot(h, wa[...], preferred_element_type=jnp.float32) + ba[...], 0.0)
    scat = jnp.dot(a, wb[...], preferred_element_type=jnp.float32) + bb[...]
    zs = scat[:, :64]
    zt = scat[:, 64:]
    p = jnp.dot(zs, zt.T, preferred_element_type=jnp.float32)
    r = lax.broadcasted_iota(jnp.int32, (t2, t2), 0)
    cc = lax.broadcasted_iota(jnp.int32, (t2, t2), 1)
    pm = jnp.where((r // 9) == (cc // 9), p, 0.0)
    kc = lax.broadcasted_iota(jnp.int32, (t2, 16), 0) % 9
    kk = lax.broadcasted_iota(jnp.int32, (t2, 16), 1)
    fold = (kc == kk).astype(jnp.float32)
    o_ref[...] = jnp.dot(pm, fold, preferred_element_type=jnp.float32)


def _blockdiag(a, b):
    z = jnp.zeros((a.shape[0] + b.shape[0], a.shape[1] + b.shape[1]), jnp.float32)
    return z.at[:a.shape[0], :a.shape[1]].set(a).at[a.shape[0]:, a.shape[1]:].set(b)


def kernel(x, edge_index, W1a, b1a, W1b, b1b, W2a, b2a, W2b, b2b,
           W3a, b3a, W3b, b3b, W4a, b4a, W4b, b4b):
    n, in_dim = x.shape
    e = edge_index.shape[1]
    ept = -(-e // (NT * KB * B)) * (KB * B)
    e_pad = NT * ept

    src = edge_index[0]
    dst = edge_index[1]
    pad = e_pad - e
    srcn2 = jnp.concatenate([src * 2, jnp.zeros((pad,), jnp.int32)])
    srcn8 = jnp.concatenate([src * 8, jnp.zeros((pad,), jnp.int32)])
    dstp = jnp.concatenate([dst, jnp.full((pad,), n, jnp.int32)])
    zrows = jnp.zeros((N_ACC, 16), jnp.float32)

    x32 = jnp.pad(x, ((0, 0), (0, 32 - in_dim)))
    aggx = _make_sc_agg(2, n, ept)(
        x32.reshape(n * 2, 16), srcn2, dstp, zrows).reshape(N_ACC, 32)

    wa1 = jnp.concatenate(
        [jnp.pad(W1a, ((0, 2), (0, 0))), jnp.pad(W3a, ((0, 2), (0, 0)))], axis=1)
    ba1 = jnp.concatenate([b1a, b3a]).reshape(1, 128)
    wb1 = _blockdiag(W1b, W3b)
    bb1 = jnp.concatenate([b1b, b3b]).reshape(1, 128)
    t1 = 512
    g1 = -(-n // t1)
    zcat = pl.pallas_call(
        _tc_mlp,
        grid=(g1,),
        in_specs=[
            pl.BlockSpec((t1, 32), lambda i: (i, 0)),
            pl.BlockSpec((t1, 32), lambda i: (i, 0)),
            pl.BlockSpec((32, 128), lambda i: (0, 0)),
            pl.BlockSpec((1, 128), lambda i: (0, 0)),
            pl.BlockSpec((128, 128), lambda i: (0, 0)),
            pl.BlockSpec((1, 128), lambda i: (0, 0)),
        ],
        out_specs=pl.BlockSpec((t1, 128), lambda i: (i, 0)),
        out_shape=jax.ShapeDtypeStruct((n, 128), jnp.float32),
    )(x32, aggx, wa1, ba1, wb1, bb1)

    aggz = _make_sc_agg(8, n, ept)(
        zcat.reshape(n * 8, 16), srcn8, dstp, zrows).reshape(N_ACC, 128)

    wa2 = _blockdiag(W2a, W4a)
    ba2 = jnp.concatenate([b2a, b4a]).reshape(1, 128)
    wb2 = _blockdiag(W2b, W4b)
    bb2 = jnp.concatenate([b2b, b4b]).reshape(1, 128)
    t2 = 576
    g2 = -(-n // t2)
    out16 = pl.pallas_call(
        functools.partial(_tc_mlp_dec, t2),
        grid=(g2,),
        in_specs=[
            pl.BlockSpec((t2, 128), lambda i: (i, 0)),
            pl.BlockSpec((t2, 128), lambda i: (i, 0)),
            pl.BlockSpec((128, 128), lambda i: (0, 0)),
            pl.BlockSpec((1, 128), lambda i: (0, 0)),
            pl.BlockSpec((128, 128), lambda i: (0, 0)),
            pl.BlockSpec((1, 128), lambda i: (0, 0)),
        ],
        out_specs=pl.BlockSpec((t2, 16), lambda i: (i, 0)),
        out_shape=jax.ShapeDtypeStruct((n, 16), jnp.float32),
    )(zcat, aggz, wa2, ba2, wb2, bb2)
    return out16[:, :9]


# speedup vs baseline: 9.9314x; 1.1107x over previous
import functools

import jax
import jax.numpy as jnp
from jax import lax
from jax.experimental import pallas as pl
from jax.experimental.pallas import tpu as pltpu
from jax.experimental.pallas import tpu_sc as plsc

NT = 16
NC = 2
B = 128
KB = 6
N_ACC = 100096


def _make_sc_agg(nchunk, n_rows, ept):
    cps = nchunk // NC
    gsz = KB * B
    ngroups = ept // gsz
    half = ngroups // 2
    rpt = N_ACC // NT
    lview = n_rows * nchunk - nchunk + 1
    scratch = (
        [pltpu.VMEM((B,), jnp.int32) for _ in range(4 * KB)]
        + [pltpu.VMEM((B, 16), jnp.float32) for _ in range(2 * KB)]
        + [pltpu.VMEM_SHARED((N_ACC, 16), jnp.float32)]
        + [pltpu.SemaphoreType.DMA for _ in range(4)]
    )
    mesh = plsc.VectorSubcoreMesh(core_axis_name="c", subcore_axis_name="s")

    @functools.partial(
        pl.kernel, mesh=mesh,
        out_type=jax.ShapeDtypeStruct((N_ACC, nchunk, 16), jnp.float32),
        scratch_types=scratch,
        compiler_params=pltpu.CompilerParams(use_tc_tiling_on_sc=False))
    def sc_agg(tables, srcn, dstp, zrows, out, *rest):
        srcv = (rest[0:KB], rest[KB:2 * KB])
        dstv = (rest[2 * KB:3 * KB], rest[3 * KB:4 * KB])
        rowv = (rest[4 * KB:5 * KB], rest[5 * KB:6 * KB])
        acc = rest[6 * KB]
        sem_i = rest[6 * KB + 1:6 * KB + 3]
        sem_g = rest[6 * KB + 3:6 * KB + 5]
        c = lax.axis_index("c")
        s = lax.axis_index("s")
        row0 = s * rpt
        ebase = s * ept
        maxbase = ebase + ept - gsz

        def fire_idx(p, base):
            for b in range(KB):
                o = pl.multiple_of(base + b * B, B)
                pltpu.async_copy(srcn.at[pl.ds(o, B)], srcv[p][b], sem_i[p])
                pltpu.async_copy(dstp.at[pl.ds(o, B)], dstv[p][b], sem_i[p])

        def wait_idx(p):
            for b in range(KB):
                pltpu.make_async_copy(srcn.at[pl.ds(0, B)], srcv[p][b], sem_i[p]).wait()
                pltpu.make_async_copy(dstp.at[pl.ds(0, B)], dstv[p][b], sem_i[p]).wait()

        for core_id in range(NC):
            @pl.when(c == core_id)
            def _():
                for k in range(cps):
                    chunk = core_id * cps + k
                    tview = tables.at[pl.ds(chunk, lview)]

                    def fire_gather(p):
                        for b in range(KB):
                            pltpu.async_copy(tview.at[srcv[p][b]], rowv[p][b],
                                             sem_g[p])

                    def wait_gather(p):
                        for b in range(KB):
                            pltpu.make_async_copy(tview.at[srcv[p][b]],
                                                  rowv[p][b], sem_g[p]).wait()

                    def scatter(p):
                        for b in range(KB):
                            pltpu.sync_copy(rowv[p][b], acc.at[dstv[p][b]],
                                            add=True)

                    pltpu.sync_copy(zrows.at[pl.ds(row0, rpt)],
                                    acc.at[pl.ds(row0, rpt)])
                    plsc.subcore_barrier()

                    fire_idx(0, ebase)
                    wait_idx(0)
                    fire_gather(0)
                    fire_idx(1, ebase + gsz)

                    def body(i, carry):
                        base_a = jnp.minimum(ebase + (2 * i + 2) * gsz, maxbase)
                        base_b = jnp.minimum(ebase + (2 * i + 3) * gsz, maxbase)
                        wait_gather(0)
                        wait_idx(1)
                        fire_gather(1)
                        scatter(0)
                        fire_idx(0, base_a)
                        wait_gather(1)
                        wait_idx(0)
                        fire_gather(0)
                        scatter(1)
                        fire_idx(1, base_b)
                        return carry

                    lax.fori_loop(0, half, body, 0)
                    wait_gather(0)
                    wait_idx(1)
                    plsc.subcore_barrier()
                    pltpu.sync_copy(acc.at[pl.ds(row0, rpt)],
                                    out.at[pl.ds(row0, rpt), chunk])
                    plsc.subcore_barrier()

    return sc_agg


def _tc_mlp(x_ref, g_ref, wa, ba, wb, bb, o_ref):
    h = x_ref[...] + g_ref[...]
    a = jnp.maximum(jnp.dot(h, wa[...], preferred_element_type=jnp.float32) + ba[...], 0.0)
    o_ref[...] = jnp.dot(a, wb[...], preferred_element_type=jnp.float32) + bb[...]


def _tc_mlp_dec(t2, z_ref, g_ref, wa, ba, wb, bb, o_ref):
    h = z_ref[...] + g_ref[...]
    a = jnp.maximum(jnp.dot(h, wa[...], preferred_element_type=jnp.float32) + ba[...], 0.0)
    scat = jnp.dot(a, wb[...], preferred_element_type=jnp.float32) + bb[...]
    zs = scat[:, :64]
    zt = scat[:, 64:]
    p = jnp.dot(zs, zt.T, preferred_element_type=jnp.float32)
    r = lax.broadcasted_iota(jnp.int32, (t2, t2), 0)
    cc = lax.broadcasted_iota(jnp.int32, (t2, t2), 1)
    pm = jnp.where((r // 9) == (cc // 9), p, 0.0)
    kc = lax.broadcasted_iota(jnp.int32, (t2, 16), 0) % 9
    kk = lax.broadcasted_iota(jnp.int32, (t2, 16), 1)
    fold = (kc == kk).astype(jnp.float32)
    o_ref[...] = jnp.dot(pm, fold, preferred_element_type=jnp.float32)


def _blockdiag(a, b):
    z = jnp.zeros((a.shape[0] + b.shape[0], a.shape[1] + b.shape[1]), jnp.float32)
    return z.at[:a.shape[0], :a.shape[1]].set(a).at[a.shape[0]:, a.shape[1]:].set(b)


def kernel(x, edge_index, W1a, b1a, W1b, b1b, W2a, b2a, W2b, b2b,
           W3a, b3a, W3b, b3b, W4a, b4a, W4b, b4b):
    n, in_dim = x.shape
    e = edge_index.shape[1]
    ept = -(-e // (NT * 2 * KB * B)) * (2 * KB * B)
    e_pad = NT * ept

    src = edge_index[0]
    dst = edge_index[1]
    pad = e_pad - e
    srcn2 = jnp.concatenate([src * 2, jnp.zeros((pad,), jnp.int32)])
    srcn8 = jnp.concatenate([src * 8, jnp.zeros((pad,), jnp.int32)])
    dstp = jnp.concatenate([dst, jnp.full((pad,), n, jnp.int32)])
    zrows = jnp.zeros((N_ACC, 16), jnp.float32)

    x32 = jnp.pad(x, ((0, 0), (0, 32 - in_dim)))
    aggx = _make_sc_agg(2, n, ept)(
        x32.reshape(n * 2, 16), srcn2, dstp, zrows).reshape(N_ACC, 32)

    wa1 = jnp.concatenate(
        [jnp.pad(W1a, ((0, 2), (0, 0))), jnp.pad(W3a, ((0, 2), (0, 0)))], axis=1)
    ba1 = jnp.concatenate([b1a, b3a]).reshape(1, 128)
    wb1 = _blockdiag(W1b, W3b)
    bb1 = jnp.concatenate([b1b, b3b]).reshape(1, 128)
    t1 = 512
    g1 = -(-n // t1)
    zcat = pl.pallas_call(
        _tc_mlp,
        grid=(g1,),
        in_specs=[
            pl.BlockSpec((t1, 32), lambda i: (i, 0)),
            pl.BlockSpec((t1, 32), lambda i: (i, 0)),
            pl.BlockSpec((32, 128), lambda i: (0, 0)),
            pl.BlockSpec((1, 128), lambda i: (0, 0)),
            pl.BlockSpec((128, 128), lambda i: (0, 0)),
            pl.BlockSpec((1, 128), lambda i: (0, 0)),
        ],
        out_specs=pl.BlockSpec((t1, 128), lambda i: (i, 0)),
        out_shape=jax.ShapeDtypeStruct((n, 128), jnp.float32),
    )(x32, aggx, wa1, ba1, wb1, bb1)

    aggz = _make_sc_agg(8, n, ept)(
        zcat.reshape(n * 8, 16), srcn8, dstp, zrows).reshape(N_ACC, 128)

    wa2 = _blockdiag(W2a, W4a)
    ba2 = jnp.concatenate([b2a, b4a]).reshape(1, 128)
    wb2 = _blockdiag(W2b, W4b)
    bb2 = jnp.concatenate([b2b, b4b]).reshape(1, 128)
    t2 = 576
    g2 = -(-n // t2)
    out16 = pl.pallas_call(
        functools.partial(_tc_mlp_dec, t2),
        grid=(g2,),
        in_specs=[
            pl.BlockSpec((t2, 128), lambda i: (i, 0)),
            pl.BlockSpec((t2, 128), lambda i: (i, 0)),
            pl.BlockSpec((128, 128), lambda i: (0, 0)),
            pl.BlockSpec((1, 128), lambda i: (0, 0)),
            pl.BlockSpec((128, 128), lambda i: (0, 0)),
            pl.BlockSpec((1, 128), lambda i: (0, 0)),
        ],
        out_specs=pl.BlockSpec((t2, 16), lambda i: (i, 0)),
        out_shape=jax.ShapeDtypeStruct((n, 16), jnp.float32),
    )(zcat, aggz, wa2, ba2, wb2, bb2)
    return out16[:, :9]


# speedup vs baseline: 11.7100x; 1.1791x over previous
import functools

import jax
import jax.numpy as jnp
from jax import lax
from jax.experimental import pallas as pl
from jax.experimental.pallas import tpu as pltpu
from jax.experimental.pallas import tpu_sc as plsc

NT = 16
NC = 2
B = 128
KB = 4
NSET = 3
N_ACC = 100096


def _make_sc_agg(nchunk, n_rows, ept):
    cps = nchunk // NC
    gsz = KB * B
    ngroups = ept // gsz
    assert ngroups % 3 == 1 and ngroups >= 4
    m = (ngroups - 1) // 3
    rpt = N_ACC // NT
    lview = n_rows * nchunk - nchunk + 1
    scratch = (
        [pltpu.VMEM((2, B), jnp.int32) for _ in range(NSET * KB)]
        + [pltpu.VMEM((B, 16), jnp.float32) for _ in range(NSET * KB)]
        + [pltpu.VMEM_SHARED((N_ACC, 16), jnp.float32)]
        + [pltpu.SemaphoreType.DMA for _ in range(3 * NSET)]
    )
    mesh = plsc.VectorSubcoreMesh(core_axis_name="c", subcore_axis_name="s")

    @functools.partial(
        pl.kernel, mesh=mesh,
        out_type=jax.ShapeDtypeStruct((N_ACC, nchunk, 16), jnp.float32),
        scratch_types=scratch,
        compiler_params=pltpu.CompilerParams(use_tc_tiling_on_sc=False))
    def sc_agg(tables, epk, zrows, out, *rest):
        idxv = [rest[p * KB:(p + 1) * KB] for p in range(NSET)]
        rowv = [rest[NSET * KB + p * KB:NSET * KB + (p + 1) * KB]
                for p in range(NSET)]
        acc = rest[2 * NSET * KB]
        sems = rest[2 * NSET * KB + 1:]
        sem_i = sems[0:NSET]
        sem_g = sems[NSET:2 * NSET]
        sem_s = sems[2 * NSET:3 * NSET]
        c = lax.axis_index("c")
        s = lax.axis_index("s")
        row0 = s * rpt
        gbase = s * (ept // B)
        maxg = gbase + ept // B - KB

        def fire_idx(p, gb):
            for b in range(KB):
                pltpu.async_copy(epk.at[pl.ds((gb + b) * 2, 2)], idxv[p][b],
                                 sem_i[p])

        def wait_idx(p):
            for b in range(KB):
                pltpu.make_async_copy(epk.at[pl.ds(0, 2)], idxv[p][b],
                                      sem_i[p]).wait()

        for core_id in range(NC):
            @pl.when(c == core_id)
            def _():
                for k in range(cps):
                    chunk = core_id * cps + k
                    tview = tables.at[pl.ds(chunk, lview)]

                    def fire_gather(p):
                        for b in range(KB):
                            pltpu.async_copy(tview.at[idxv[p][b].at[0]],
                                             rowv[p][b], sem_g[p])

                    def wait_gather(p):
                        for b in range(KB):
                            pltpu.make_async_copy(tview.at[idxv[p][b].at[0]],
                                                  rowv[p][b], sem_g[p]).wait()

                    def fire_scatter(p):
                        for b in range(KB):
                            pltpu.async_copy(rowv[p][b],
                                             acc.at[idxv[p][b].at[1]],
                                             sem_s[p], add=True)

                    def wait_scatter(p):
                        for b in range(KB):
                            pltpu.make_async_copy(rowv[p][b],
                                                  acc.at[idxv[p][b].at[1]],
                                                  sem_s[p]).wait()

                    pltpu.sync_copy(zrows.at[pl.ds(row0, rpt)],
                                    acc.at[pl.ds(row0, rpt)])
                    plsc.subcore_barrier()

                    fire_idx(0, gbase)
                    fire_idx(1, gbase + KB)
                    wait_idx(0)
                    fire_gather(0)
                    wait_gather(0)
                    wait_idx(1)
                    fire_gather(1)
                    fire_idx(2, gbase + 2 * KB)
                    fire_scatter(0)

                    def body(i, carry):
                        for (p, q, r, gn) in ((1, 2, 0, 3), (2, 0, 1, 4),
                                              (0, 1, 2, 5)):
                            gnext = jnp.minimum(gbase + (3 * i + gn) * KB, maxg)
                            wait_gather(p)
                            wait_idx(q)
                            fire_gather(q)
                            wait_scatter(r)
                            fire_idx(r, gnext)
                            fire_scatter(p)
                        return carry

                    lax.fori_loop(0, m, body, 0)
                    wait_scatter(0)
                    wait_gather(1)
                    wait_idx(2)
                    plsc.subcore_barrier()
                    pltpu.sync_copy(acc.at[pl.ds(row0, rpt)],
                                    out.at[pl.ds(row0, rpt), chunk])
                    plsc.subcore_barrier()

    return sc_agg


def _tc_mlp(x_ref, g_ref, wa, ba, wb, bb, o_ref):
    h = x_ref[...] + g_ref[...]
    a = jnp.maximum(jnp.dot(h, wa[...], preferred_element_type=jnp.float32) + ba[...], 0.0)
    o_ref[...] = jnp.dot(a, wb[...], preferred_element_type=jnp.float32) + bb[...]


def _tc_mlp_dec(t2, z_ref, g_ref, wa, ba, wb, bb, o_ref):
    h = z_ref[...] + g_ref[...]
    a = jnp.maximum(jnp.dot(h, wa[...], preferred_element_type=jnp.float32) + ba[...], 0.0)
    scat = jnp.dot(a, wb[...], preferred_element_type=jnp.float32) + bb[...]
    zs = scat[:, :64]
    zt = scat[:, 64:]
    p = jnp.dot(zs, zt.T, preferred_element_type=jnp.float32)
    r = lax.broadcasted_iota(jnp.int32, (t2, t2), 0)
    cc = lax.broadcasted_iota(jnp.int32, (t2, t2), 1)
    pm = jnp.where((r // 9) == (cc // 9), p, 0.0)
    kc = lax.broadcasted_iota(jnp.int32, (t2, 16), 0) % 9
    kk = lax.broadcasted_iota(jnp.int32, (t2, 16), 1)
    fold = (kc == kk).astype(jnp.float32)
    o_ref[...] = jnp.dot(pm, fold, preferred_element_type=jnp.float32)


def _blockdiag(a, b):
    z = jnp.zeros((a.shape[0] + b.shape[0], a.shape[1] + b.shape[1]), jnp.float32)
    return z.at[:a.shape[0], :a.shape[1]].set(a).at[a.shape[0]:, a.shape[1]:].set(b)


def _pack_edges(srcn, dstp):
    return jnp.stack([srcn.reshape(-1, B), dstp.reshape(-1, B)], 1).reshape(-1, B)


def kernel(x, edge_index, W1a, b1a, W1b, b1b, W2a, b2a, W2b, b2b,
           W3a, b3a, W3b, b3b, W4a, b4a, W4b, b4b):
    n, in_dim = x.shape
    e = edge_index.shape[1]
    gsz = KB * B
    ngroups = -(-e // (NT * gsz))
    if ngroups % 3 != 1:
        ngroups += (1 - ngroups) % 3
    ept = ngroups * gsz
    e_pad = NT * ept

    src = edge_index[0]
    dst = edge_index[1]
    pad = e_pad - e
    dstp = jnp.concatenate([dst, jnp.full((pad,), n, jnp.int32)])
    zpad = jnp.zeros((pad,), jnp.int32)
    epk2 = _pack_edges(jnp.concatenate([src * 2, zpad]), dstp)
    epk8 = _pack_edges(jnp.concatenate([src * 8, zpad]), dstp)
    zrows = jnp.zeros((N_ACC, 16), jnp.float32)

    x32 = jnp.pad(x, ((0, 0), (0, 32 - in_dim)))
    aggx = _make_sc_agg(2, n, ept)(
        x32.reshape(n * 2, 16), epk2, zrows).reshape(N_ACC, 32)

    wa1 = jnp.concatenate(
        [jnp.pad(W1a, ((0, 2), (0, 0))), jnp.pad(W3a, ((0, 2), (0, 0)))], axis=1)
    ba1 = jnp.concatenate([b1a, b3a]).reshape(1, 128)
    wb1 = _blockdiag(W1b, W3b)
    bb1 = jnp.concatenate([b1b, b3b]).reshape(1, 128)
    t1 = 512
    g1 = -(-n // t1)
    zcat = pl.pallas_call(
        _tc_mlp,
        grid=(g1,),
        in_specs=[
            pl.BlockSpec((t1, 32), lambda i: (i, 0)),
            pl.BlockSpec((t1, 32), lambda i: (i, 0)),
            pl.BlockSpec((32, 128), lambda i: (0, 0)),
            pl.BlockSpec((1, 128), lambda i: (0, 0)),
            pl.BlockSpec((128, 128), lambda i: (0, 0)),
            pl.BlockSpec((1, 128), lambda i: (0, 0)),
        ],
        out_specs=pl.BlockSpec((t1, 128), lambda i: (i, 0)),
        out_shape=jax.ShapeDtypeStruct((n, 128), jnp.float32),
    )(x32, aggx, wa1, ba1, wb1, bb1)

    aggz = _make_sc_agg(8, n, ept)(
        zcat.reshape(n * 8, 16), epk8, zrows).reshape(N_ACC, 128)

    wa2 = _blockdiag(W2a, W4a)
    ba2 = jnp.concatenate([b2a, b4a]).reshape(1, 128)
    wb2 = _blockdiag(W2b, W4b)
    bb2 = jnp.concatenate([b2b, b4b]).reshape(1, 128)
    t2 = 576
    g2 = -(-n // t2)
    out16 = pl.pallas_call(
        functools.partial(_tc_mlp_dec, t2),
        grid=(g2,),
        in_specs=[
            pl.BlockSpec((t2, 128), lambda i: (i, 0)),
            pl.BlockSpec((t2, 128), lambda i: (i, 0)),
            pl.BlockSpec((128, 128), lambda i: (0, 0)),
            pl.BlockSpec((1, 128), lambda i: (0, 0)),
            pl.BlockSpec((128, 128), lambda i: (0, 0)),
            pl.BlockSpec((1, 128), lambda i: (0, 0)),
        ],
        out_specs=pl.BlockSpec((t2, 16), lambda i: (i, 0)),
        out_shape=jax.ShapeDtypeStruct((n, 16), jnp.float32),
    )(zcat, aggz, wa2, ba2, wb2, bb2)
    return out16[:, :9]


# speedup vs baseline: 11.7448x; 1.0030x over previous
import functools

import jax
import jax.numpy as jnp
from jax import lax
from jax.experimental import pallas as pl
from jax.experimental.pallas import tpu as pltpu
from jax.experimental.pallas import tpu_sc as plsc

NT = 16
NC = 2
B = 256
KB = 2
NSET = 3
N_ACC = 100096


def _make_sc_agg(nchunk, n_rows, ept):
    cps = nchunk // NC
    gsz = KB * B
    ngroups = ept // gsz
    assert ngroups % 3 == 1 and ngroups >= 4
    m = (ngroups - 1) // 3
    rpt = N_ACC // NT
    lview = n_rows * nchunk - nchunk + 1
    scratch = (
        [pltpu.VMEM((2, B), jnp.int32) for _ in range(NSET * KB)]
        + [pltpu.VMEM((B, 16), jnp.float32) for _ in range(NSET * KB)]
        + [pltpu.VMEM_SHARED((N_ACC, 16), jnp.float32)]
        + [pltpu.SemaphoreType.DMA for _ in range(3 * NSET)]
    )
    mesh = plsc.VectorSubcoreMesh(core_axis_name="c", subcore_axis_name="s")

    @functools.partial(
        pl.kernel, mesh=mesh,
        out_type=jax.ShapeDtypeStruct((N_ACC, nchunk, 16), jnp.float32),
        scratch_types=scratch,
        compiler_params=pltpu.CompilerParams(use_tc_tiling_on_sc=False))
    def sc_agg(tables, epk, zrows, out, *rest):
        idxv = [rest[p * KB:(p + 1) * KB] for p in range(NSET)]
        rowv = [rest[NSET * KB + p * KB:NSET * KB + (p + 1) * KB]
                for p in range(NSET)]
        acc = rest[2 * NSET * KB]
        sems = rest[2 * NSET * KB + 1:]
        sem_i = sems[0:NSET]
        sem_g = sems[NSET:2 * NSET]
        sem_s = sems[2 * NSET:3 * NSET]
        c = lax.axis_index("c")
        s = lax.axis_index("s")
        row0 = s * rpt
        gbase = s * (ept // B)
        maxg = gbase + ept // B - KB

        def fire_idx(p, gb):
            for b in range(KB):
                pltpu.async_copy(epk.at[pl.ds((gb + b) * 2, 2)], idxv[p][b],
                                 sem_i[p])

        def wait_idx(p):
            for b in range(KB):
                pltpu.make_async_copy(epk.at[pl.ds(0, 2)], idxv[p][b],
                                      sem_i[p]).wait()

        for core_id in range(NC):
            @pl.when(c == core_id)
            def _():
                for k in range(cps):
                    chunk = core_id * cps + k
                    tview = tables.at[pl.ds(chunk, lview)]

                    def fire_gather(p):
                        for b in range(KB):
                            pltpu.async_copy(tview.at[idxv[p][b].at[0]],
                                             rowv[p][b], sem_g[p])

                    def wait_gather(p):
                        for b in range(KB):
                            pltpu.make_async_copy(tview.at[idxv[p][b].at[0]],
                                                  rowv[p][b], sem_g[p]).wait()

                    def fire_scatter(p):
                        for b in range(KB):
                            pltpu.async_copy(rowv[p][b],
                                             acc.at[idxv[p][b].at[1]],
                                             sem_s[p], add=True)

                    def wait_scatter(p):
                        for b in range(KB):
                            pltpu.make_async_copy(rowv[p][b],
                                                  acc.at[idxv[p][b].at[1]],
                                                  sem_s[p]).wait()

                    pltpu.sync_copy(zrows.at[pl.ds(row0, rpt)],
                                    acc.at[pl.ds(row0, rpt)])
                    plsc.subcore_barrier()

                    fire_idx(0, gbase)
                    fire_idx(1, gbase + KB)
                    wait_idx(0)
                    fire_gather(0)
                    wait_gather(0)
                    wait_idx(1)
                    fire_gather(1)
                    fire_idx(2, gbase + 2 * KB)
                    fire_scatter(0)

                    def body(i, carry):
                        for (p, q, r, gn) in ((1, 2, 0, 3), (2, 0, 1, 4),
                                              (0, 1, 2, 5)):
                            gnext = jnp.minimum(gbase + (3 * i + gn) * KB, maxg)
                            wait_gather(p)
                            wait_idx(q)
                            fire_gather(q)
                            wait_scatter(r)
                            fire_idx(r, gnext)
                            fire_scatter(p)
                        return carry

                    lax.fori_loop(0, m, body, 0)
                    wait_scatter(0)
                    wait_gather(1)
                    wait_idx(2)
                    plsc.subcore_barrier()
                    pltpu.sync_copy(acc.at[pl.ds(row0, rpt)],
                                    out.at[pl.ds(row0, rpt), chunk])
                    plsc.subcore_barrier()

    return sc_agg


def _tc_mlp(x_ref, g_ref, wa, ba, wb, bb, o_ref):
    h = x_ref[...] + g_ref[...]
    a = jnp.maximum(jnp.dot(h, wa[...], preferred_element_type=jnp.float32) + ba[...], 0.0)
    o_ref[...] = jnp.dot(a, wb[...], preferred_element_type=jnp.float32) + bb[...]


def _tc_mlp_dec(t2, z_ref, g_ref, wa, ba, wb, bb, o_ref):
    h = z_ref[...] + g_ref[...]
    a = jnp.maximum(jnp.dot(h, wa[...], preferred_element_type=jnp.float32) + ba[...], 0.0)
    scat = jnp.dot(a, wb[...], preferred_element_type=jnp.float32) + bb[...]
    zs = scat[:, :64]
    zt = scat[:, 64:]
    p = jnp.dot(zs, zt.T, preferred_element_type=jnp.float32)
    r = lax.broadcasted_iota(jnp.int32, (t2, t2), 0)
    cc = lax.broadcasted_iota(jnp.int32, (t2, t2), 1)
    pm = jnp.where((r // 9) == (cc // 9), p, 0.0)
    kc = lax.broadcasted_iota(jnp.int32, (t2, 16), 0) % 9
    kk = lax.broadcasted_iota(jnp.int32, (t2, 16), 1)
    fold = (kc == kk).astype(jnp.float32)
    o_ref[...] = jnp.dot(pm, fold, preferred_element_type=jnp.float32)


def _blockdiag(a, b):
    z = jnp.zeros((a.shape[0] + b.shape[0], a.shape[1] + b.shape[1]), jnp.float32)
    return z.at[:a.shape[0], :a.shape[1]].set(a).at[a.shape[0]:, a.shape[1]:].set(b)


def _pack_edges(srcn, dstp):
    return jnp.stack([srcn.reshape(-1, B), dstp.reshape(-1, B)], 1).reshape(-1, B)


def kernel(x, edge_index, W1a, b1a, W1b, b1b, W2a, b2a, W2b, b2b,
           W3a, b3a, W3b, b3b, W4a, b4a, W4b, b4b):
    n, in_dim = x.shape
    e = edge_index.shape[1]
    gsz = KB * B
    ngroups = -(-e // (NT * gsz))
    if ngroups % 3 != 1:
        ngroups += (1 - ngroups) % 3
    ept = ngroups * gsz
    e_pad = NT * ept

    src = edge_index[0]
    dst = edge_index[1]
    pad = e_pad - e
    dstp = jnp.concatenate([dst, jnp.full((pad,), n, jnp.int32)])
    zpad = jnp.zeros((pad,), jnp.int32)
    epk2 = _pack_edges(jnp.concatenate([src * 2, zpad]), dstp)
    epk8 = _pack_edges(jnp.concatenate([src * 8, zpad]), dstp)
    zrows = jnp.zeros((N_ACC, 16), jnp.float32)

    x32 = jnp.pad(x, ((0, 0), (0, 32 - in_dim)))
    aggx = _make_sc_agg(2, n, ept)(
        x32.reshape(n * 2, 16), epk2, zrows).reshape(N_ACC, 32)

    wa1 = jnp.concatenate(
        [jnp.pad(W1a, ((0, 2), (0, 0))), jnp.pad(W3a, ((0, 2), (0, 0)))], axis=1)
    ba1 = jnp.concatenate([b1a, b3a]).reshape(1, 128)
    wb1 = _blockdiag(W1b, W3b)
    bb1 = jnp.concatenate([b1b, b3b]).reshape(1, 128)
    t1 = 512
    g1 = -(-n // t1)
    zcat = pl.pallas_call(
        _tc_mlp,
        grid=(g1,),
        in_specs=[
            pl.BlockSpec((t1, 32), lambda i: (i, 0)),
            pl.BlockSpec((t1, 32), lambda i: (i, 0)),
            pl.BlockSpec((32, 128), lambda i: (0, 0)),
            pl.BlockSpec((1, 128), lambda i: (0, 0)),
            pl.BlockSpec((128, 128), lambda i: (0, 0)),
            pl.BlockSpec((1, 128), lambda i: (0, 0)),
        ],
        out_specs=pl.BlockSpec((t1, 128), lambda i: (i, 0)),
        out_shape=jax.ShapeDtypeStruct((n, 128), jnp.float32),
    )(x32, aggx, wa1, ba1, wb1, bb1)

    aggz = _make_sc_agg(8, n, ept)(
        zcat.reshape(n * 8, 16), epk8, zrows).reshape(N_ACC, 128)

    wa2 = _blockdiag(W2a, W4a)
    ba2 = jnp.concatenate([b2a, b4a]).reshape(1, 128)
    wb2 = _blockdiag(W2b, W4b)
    bb2 = jnp.concatenate([b2b, b4b]).reshape(1, 128)
    t2 = 576
    g2 = -(-n // t2)
    out16 = pl.pallas_call(
        functools.partial(_tc_mlp_dec, t2),
        grid=(g2,),
        in_specs=[
            pl.BlockSpec((t2, 128), lambda i: (i, 0)),
            pl.BlockSpec((t2, 128), lambda i: (i, 0)),
            pl.BlockSpec((128, 128), lambda i: (0, 0)),
            pl.BlockSpec((1, 128), lambda i: (0, 0)),
            pl.BlockSpec((128, 128), lambda i: (0, 0)),
            pl.BlockSpec((1, 128), lambda i: (0, 0)),
        ],
        out_specs=pl.BlockSpec((t2, 16), lambda i: (i, 0)),
        out_shape=jax.ShapeDtypeStruct((n, 16), jnp.float32),
    )(zcat, aggz, wa2, ba2, wb2, bb2)
    return out16[:, :9]


# speedup vs baseline: 11.9469x; 1.0172x over previous
import functools

import jax
import jax.numpy as jnp
from jax import lax
from jax.experimental import pallas as pl
from jax.experimental.pallas import tpu as pltpu
from jax.experimental.pallas import tpu_sc as plsc

NT = 16
NC = 2
B = 256
KB = 2
NSET = 3
N_ACC = 100096


def _make_sc_agg(stride, cps, n_rows, ept):
    gsz = KB * B
    ngroups = ept // gsz
    assert ngroups % 3 == 1 and ngroups >= 4
    m = (ngroups - 1) // 3
    rpt = N_ACC // NT
    lview = n_rows * stride - stride + 1
    scratch = (
        [pltpu.VMEM((2, B), jnp.int32) for _ in range(NSET * KB)]
        + [pltpu.VMEM((B, 16), jnp.float32) for _ in range(NSET * KB)]
        + [pltpu.VMEM((128, 16), jnp.float32)]
        + [pltpu.VMEM_SHARED((N_ACC, 16), jnp.float32)]
        + [pltpu.SemaphoreType.DMA for _ in range(3 * NSET)]
    )
    mesh = plsc.VectorSubcoreMesh(core_axis_name="c", subcore_axis_name="s")

    @functools.partial(
        pl.kernel, mesh=mesh,
        out_type=jax.ShapeDtypeStruct((N_ACC, NC * cps, 16), jnp.float32),
        scratch_types=scratch,
        compiler_params=pltpu.CompilerParams(use_tc_tiling_on_sc=False))
    def sc_agg(tables, epk, out, *rest):
        idxv = [rest[p * KB:(p + 1) * KB] for p in range(NSET)]
        rowv = [rest[NSET * KB + p * KB:NSET * KB + (p + 1) * KB]
                for p in range(NSET)]
        zbuf = rest[2 * NSET * KB]
        acc = rest[2 * NSET * KB + 1]
        sems = rest[2 * NSET * KB + 2:]
        sem_i = sems[0:NSET]
        sem_g = sems[NSET:2 * NSET]
        sem_s = sems[2 * NSET:3 * NSET]
        c = lax.axis_index("c")
        s = lax.axis_index("s")
        row0 = s * rpt
        gbase = s * (ept // B)
        maxg = gbase + ept // B - KB

        def fire_idx(p, gb):
            for b in range(KB):
                pltpu.async_copy(epk.at[pl.ds((gb + b) * 2, 2)], idxv[p][b],
                                 sem_i[p])

        def wait_idx(p):
            for b in range(KB):
                pltpu.make_async_copy(epk.at[pl.ds(0, 2)], idxv[p][b],
                                      sem_i[p]).wait()

        for j in range(128):
            zbuf[j, :] = jnp.zeros((16,), jnp.float32)
        nzc = rpt // 128
        ztail = rpt - nzc * 128

        def zero_acc():
            for j in range(nzc):
                pltpu.async_copy(zbuf, acc.at[pl.ds(row0 + j * 128, 128)],
                                 sem_g[0])
            if ztail:
                pltpu.async_copy(zbuf.at[pl.ds(0, ztail)],
                                 acc.at[pl.ds(row0 + nzc * 128, ztail)],
                                 sem_g[0])
            for j in range(nzc):
                pltpu.make_async_copy(zbuf, acc.at[pl.ds(row0, 128)],
                                      sem_g[0]).wait()
            if ztail:
                pltpu.make_async_copy(zbuf.at[pl.ds(0, ztail)],
                                      acc.at[pl.ds(row0, ztail)],
                                      sem_g[0]).wait()

        for core_id in range(NC):
            @pl.when(c == core_id)
            def _():
                for k in range(cps):
                    chunk = core_id * cps + k
                    tview = tables.at[pl.ds(chunk, lview)]

                    def fire_gather(p):
                        for b in range(KB):
                            pltpu.async_copy(tview.at[idxv[p][b].at[0]],
                                             rowv[p][b], sem_g[p])

                    def wait_gather(p):
                        for b in range(KB):
                            pltpu.make_async_copy(tview.at[idxv[p][b].at[0]],
                                                  rowv[p][b], sem_g[p]).wait()

                    def fire_scatter(p):
                        for b in range(KB):
                            pltpu.async_copy(rowv[p][b],
                                             acc.at[idxv[p][b].at[1]],
                                             sem_s[p], add=True)

                    def wait_scatter(p):
                        for b in range(KB):
                            pltpu.make_async_copy(rowv[p][b],
                                                  acc.at[idxv[p][b].at[1]],
                                                  sem_s[p]).wait()

                    zero_acc()
                    plsc.subcore_barrier()

                    fire_idx(0, gbase)
                    fire_idx(1, gbase + KB)
                    wait_idx(0)
                    fire_gather(0)
                    wait_gather(0)
                    wait_idx(1)
                    fire_gather(1)
                    fire_idx(2, gbase + 2 * KB)
                    fire_scatter(0)

                    def body(i, carry):
                        for (p, q, r, gn) in ((1, 2, 0, 3), (2, 0, 1, 4),
                                              (0, 1, 2, 5)):
                            gnext = jnp.minimum(gbase + (3 * i + gn) * KB, maxg)
                            wait_gather(p)
                            wait_idx(q)
                            fire_gather(q)
                            wait_scatter(r)
                            fire_idx(r, gnext)
                            fire_scatter(p)
                        return carry

                    lax.fori_loop(0, m, body, 0)
                    wait_scatter(0)
                    wait_gather(1)
                    wait_idx(2)
                    plsc.subcore_barrier()
                    pltpu.sync_copy(acc.at[pl.ds(row0, rpt)],
                                    out.at[pl.ds(row0, rpt), chunk])
                    plsc.subcore_barrier()

    return sc_agg


def _tc_mlp(x_ref, g_ref, wa, ba, wb, bb, o_ref):
    h = x_ref[...] + g_ref[...]
    a = jnp.maximum(jnp.dot(h, wa[...], preferred_element_type=jnp.float32) + ba[...], 0.0)
    o_ref[...] = jnp.dot(a, wb[...], preferred_element_type=jnp.float32) + bb[...]


def _tc_mlp_dec(t2, z_ref, g_ref, wa, ba, wb, bb, o_ref):
    h = z_ref[...] + g_ref[...]
    a = jnp.maximum(jnp.dot(h, wa[...], preferred_element_type=jnp.float32) + ba[...], 0.0)
    scat = jnp.dot(a, wb[...], preferred_element_type=jnp.float32) + bb[...]
    zs = scat[:, :64]
    zt = scat[:, 64:]
    p = jnp.dot(zs, zt.T, preferred_element_type=jnp.float32)
    r = lax.broadcasted_iota(jnp.int32, (t2, t2), 0)
    cc = lax.broadcasted_iota(jnp.int32, (t2, t2), 1)
    pm = jnp.where((r // 9) == (cc // 9), p, 0.0)
    kc = lax.broadcasted_iota(jnp.int32, (t2, 16), 0) % 9
    kk = lax.broadcasted_iota(jnp.int32, (t2, 16), 1)
    fold = (kc == kk).astype(jnp.float32)
    o_ref[...] = jnp.dot(pm, fold, preferred_element_type=jnp.float32)


def _blockdiag(a, b):
    z = jnp.zeros((a.shape[0] + b.shape[0], a.shape[1] + b.shape[1]), jnp.float32)
    return z.at[:a.shape[0], :a.shape[1]].set(a).at[a.shape[0]:, a.shape[1]:].set(b)


def _pack_edges(srcn, dstp):
    return jnp.stack([srcn.reshape(-1, B), dstp.reshape(-1, B)], 1).reshape(-1, B)


def kernel(x, edge_index, W1a, b1a, W1b, b1b, W2a, b2a, W2b, b2b,
           W3a, b3a, W3b, b3b, W4a, b4a, W4b, b4b):
    n, in_dim = x.shape
    e = edge_index.shape[1]
    gsz = KB * B
    ngroups = -(-e // (NT * gsz))
    if ngroups % 3 != 1:
        ngroups += (1 - ngroups) % 3
    ept = ngroups * gsz
    e_pad = NT * ept

    src = edge_index[0]
    dst = edge_index[1]
    pad = e_pad - e
    dstp = jnp.concatenate([dst, jnp.full((pad,), n, jnp.int32)])
    zpad = jnp.zeros((pad,), jnp.int32)
    epk8 = _pack_edges(jnp.concatenate([src * 8, zpad]), dstp)

    x32 = jnp.pad(x, ((0, 0), (0, 32 - in_dim)))
    x128 = jnp.pad(x, ((0, 0), (0, 128 - in_dim)))
    aggx = _make_sc_agg(8, 1, n, ept)(
        x128.reshape(n * 8, 16), epk8).reshape(N_ACC, 32)

    wa1 = jnp.concatenate(
        [jnp.pad(W1a, ((0, 2), (0, 0))), jnp.pad(W3a, ((0, 2), (0, 0)))], axis=1)
    ba1 = jnp.concatenate([b1a, b3a]).reshape(1, 128)
    wb1 = _blockdiag(W1b, W3b)
    bb1 = jnp.concatenate([b1b, b3b]).reshape(1, 128)
    t1 = 512
    g1 = -(-n // t1)
    zcat = pl.pallas_call(
        _tc_mlp,
        grid=(g1,),
        in_specs=[
            pl.BlockSpec((t1, 32), lambda i: (i, 0)),
            pl.BlockSpec((t1, 32), lambda i: (i, 0)),
            pl.BlockSpec((32, 128), lambda i: (0, 0)),
            pl.BlockSpec((1, 128), lambda i: (0, 0)),
            pl.BlockSpec((128, 128), lambda i: (0, 0)),
            pl.BlockSpec((1, 128), lambda i: (0, 0)),
        ],
        out_specs=pl.BlockSpec((t1, 128), lambda i: (i, 0)),
        out_shape=jax.ShapeDtypeStruct((n, 128), jnp.float32),
    )(x32, aggx, wa1, ba1, wb1, bb1)

    aggz = _make_sc_agg(8, 4, n, ept)(
        zcat.reshape(n * 8, 16), epk8).reshape(N_ACC, 128)

    wa2 = _blockdiag(W2a, W4a)
    ba2 = jnp.concatenate([b2a, b4a]).reshape(1, 128)
    wb2 = _blockdiag(W2b, W4b)
    bb2 = jnp.concatenate([b2b, b4b]).reshape(1, 128)
    t2 = 576
    g2 = -(-n // t2)
    out16 = pl.pallas_call(
        functools.partial(_tc_mlp_dec, t2),
        grid=(g2,),
        in_specs=[
            pl.BlockSpec((t2, 128), lambda i: (i, 0)),
            pl.BlockSpec((t2, 128), lambda i: (i, 0)),
            pl.BlockSpec((128, 128), lambda i: (0, 0)),
            pl.BlockSpec((1, 128), lambda i: (0, 0)),
            pl.BlockSpec((128, 128), lambda i: (0, 0)),
            pl.BlockSpec((1, 128), lambda i: (0, 0)),
        ],
        out_specs=pl.BlockSpec((t2, 16), lambda i: (i, 0)),
        out_shape=jax.ShapeDtypeStruct((n, 16), jnp.float32),
    )(zcat, aggz, wa2, ba2, wb2, bb2)
    return out16[:, :9]
